# baseline (device time: 21694 ns/iter reference)
import jax
import jax.numpy as jnp
from jax import lax
from jax.experimental import pallas as pl
from jax.experimental.pallas import tpu as pltpu

N_DEV = 8
B = 2
SQ = 128
HQ_LOCAL = 4
DH = 64
D_MODEL = 512
D_LOC = HQ_LOCAL * DH
ROWS = B * SQ
SH = ROWS // N_DEV


def _body(x_ref, k_ref, v_ref, wq_hbm, wo_hbm, out_ref,
          part_ref, rs_ref, red_ref, ag_ref, wq_vmem, wo_vmem,
          send1, recv1, send2, recv2, load_sems):
    my = lax.axis_index("i")

    cp_wq = pltpu.make_async_copy(
        wq_hbm.at[:, pl.ds(my * D_LOC, D_LOC)], wq_vmem, load_sems.at[0])
    cp_wq.start()
    cp_wo = pltpu.make_async_copy(
        wo_hbm.at[pl.ds(my * D_LOC, D_LOC), :], wo_vmem, load_sems.at[1])
    cp_wo.start()

    barrier = pltpu.get_barrier_semaphore()
    for k in range(1, N_DEV):
        pl.semaphore_signal(barrier, inc=1,
                            device_id=(lax.rem(my + k, N_DEV),),
                            device_id_type=pl.DeviceIdType.MESH)

    row_blk = lax.broadcasted_iota(jnp.int32, (SQ, SQ), 0) // 64
    col_blk = lax.broadcasted_iota(jnp.int32, (SQ, SQ), 1) // 64
    mask = (row_blk == col_blk) | (
        lax.rem(col_blk, 4) == lax.rem(row_blk, 4))

    cp_wq.wait()
    wq_bf = wq_vmem[:].astype(jnp.bfloat16)
    x2 = x_ref[:].reshape(ROWS, D_MODEL).astype(jnp.bfloat16)
    q_all = lax.dot_general(x2, wq_bf, (((1,), (0,)), ((), ())),
                            preferred_element_type=jnp.float32)
    q_all = q_all.astype(jnp.bfloat16)

    cp_wo.wait()
    for b in range(B):
        part = jnp.zeros((SQ, D_MODEL), jnp.float32)
        for h in range(HQ_LOCAL):
            q = q_all[b * SQ:(b + 1) * SQ, h * DH:(h + 1) * DH]
            kh = k_ref[b, :, h * DH:(h + 1) * DH].astype(jnp.bfloat16)
            s = lax.dot_general(q, kh, (((1,), (1,)), ((), ())),
                                preferred_element_type=jnp.float32) * 0.125
            s = jnp.where(mask, s, -1e9)
            m = jnp.max(s, axis=1, keepdims=True)
            e = jnp.exp(s - m)
            w = (e / jnp.sum(e, axis=1, keepdims=True)).astype(jnp.bfloat16)
            vh = v_ref[b, :, h * DH:(h + 1) * DH].astype(jnp.bfloat16)
            ctx = lax.dot_general(w, vh, (((1,), (0,)), ((), ())),
                                  preferred_element_type=jnp.float32
                                  ).astype(jnp.bfloat16)
            wo_h = wo_vmem[h * DH:(h + 1) * DH, :].astype(jnp.bfloat16)
            part = part + lax.dot_general(
                ctx, wo_h, (((1,), (0,)), ((), ())),
                preferred_element_type=jnp.float32)
        part_ref[b * SQ:(b + 1) * SQ, :] = part.astype(jnp.bfloat16)

    pl.semaphore_wait(barrier, N_DEV - 1)

    sends_1 = []
    for k in range(1, N_DEV):
        peer = lax.rem(my + k, N_DEV)
        rdma = pltpu.make_async_remote_copy(
            src_ref=part_ref.at[pl.ds(peer * SH, SH)],
            dst_ref=rs_ref.at[my],
            send_sem=send1.at[k - 1],
            recv_sem=recv1.at[my],
            device_id=(peer,),
            device_id_type=pl.DeviceIdType.MESH,
        )
        rdma.start()
        sends_1.append(rdma)
    for k in range(1, N_DEV):
        src = lax.rem(my + k, N_DEV)
        pltpu.make_async_remote_copy(
            src_ref=part_ref.at[pl.ds(src * SH, SH)],
            dst_ref=rs_ref.at[src],
            send_sem=send1.at[k - 1],
            recv_sem=recv1.at[src],
            device_id=(src,),
            device_id_type=pl.DeviceIdType.MESH,
        ).wait_recv()

    red = part_ref[pl.ds(my * SH, SH), :].astype(jnp.float32)
    for o in range(N_DEV):
        scale = jnp.where(o == my, 0.0, 1.0).astype(jnp.float32)
        red = red + scale * rs_ref[o].astype(jnp.float32)
    red_ref[...] = red.astype(jnp.bfloat16)
    ag_ref[my] = red_ref[...]

    sends_2 = []
    for k in range(1, N_DEV):
        peer = lax.rem(my + k, N_DEV)
        rdma = pltpu.make_async_remote_copy(
            src_ref=red_ref,
            dst_ref=ag_ref.at[my],
            send_sem=send2.at[k - 1],
            recv_sem=recv2.at[my],
            device_id=(peer,),
            device_id_type=pl.DeviceIdType.MESH,
        )
        rdma.start()
        sends_2.append(rdma)
    for k in range(1, N_DEV):
        src = lax.rem(my + k, N_DEV)
        pltpu.make_async_remote_copy(
            src_ref=red_ref,
            dst_ref=ag_ref.at[src],
            send_sem=send2.at[k - 1],
            recv_sem=recv2.at[src],
            device_id=(src,),
            device_id_type=pl.DeviceIdType.MESH,
        ).wait_recv()

    for s in range(N_DEV):
        out_ref[s * SH // SQ, (s * SH) % SQ:(s * SH) % SQ + SH, :] = (
            ag_ref[s].astype(jnp.float32))

    for rdma in sends_1 + sends_2:
        rdma.wait_send()


def kernel(x, Wq, K_ext, V_ext, Wo):
    k2 = K_ext.reshape(B, SQ, D_LOC)
    v2 = V_ext.reshape(B, SQ, D_LOC)

    return pl.pallas_call(
        _body,
        out_shape=jax.ShapeDtypeStruct((B, SQ, D_MODEL), jnp.float32),
        in_specs=[
            pl.BlockSpec(memory_space=pltpu.VMEM),
            pl.BlockSpec(memory_space=pltpu.VMEM),
            pl.BlockSpec(memory_space=pltpu.VMEM),
            pl.BlockSpec(memory_space=pltpu.MemorySpace.HBM),
            pl.BlockSpec(memory_space=pltpu.MemorySpace.HBM),
        ],
        out_specs=pl.BlockSpec(memory_space=pltpu.VMEM),
        scratch_shapes=[
            pltpu.VMEM((ROWS, D_MODEL), jnp.bfloat16),
            pltpu.VMEM((N_DEV, SH, D_MODEL), jnp.bfloat16),
            pltpu.VMEM((SH, D_MODEL), jnp.bfloat16),
            pltpu.VMEM((N_DEV, SH, D_MODEL), jnp.bfloat16),
            pltpu.VMEM((D_MODEL, D_LOC), jnp.float32),
            pltpu.VMEM((D_LOC, D_MODEL), jnp.float32),
            pltpu.SemaphoreType.DMA((N_DEV - 1,)),
            pltpu.SemaphoreType.DMA((N_DEV,)),
            pltpu.SemaphoreType.DMA((N_DEV - 1,)),
            pltpu.SemaphoreType.DMA((N_DEV,)),
            pltpu.SemaphoreType.DMA((2,)),
        ],
        compiler_params=pltpu.CompilerParams(collective_id=0),
    )(x, k2, v2, Wq, Wo)


# device time: 18448 ns/iter; 1.1760x vs baseline; 1.1760x over previous
import jax
import jax.numpy as jnp
from jax import lax
from jax.experimental import pallas as pl
from jax.experimental.pallas import tpu as pltpu

N_DEV = 8
B = 2
SQ = 128
HQ_LOCAL = 4
DH = 64
D_MODEL = 512
D_LOC = HQ_LOCAL * DH
ROWS = B * SQ
SH = ROWS // N_DEV


def _body(x_ref, k_ref, v_ref, wq_ref, wo_ref, out_ref,
          part_ref, rs_ref, red_ref, ag_ref,
          send1, recv1, send2, recv2):
    my = lax.axis_index("i")

    barrier = pltpu.get_barrier_semaphore()
    for k in range(1, N_DEV):
        pl.semaphore_signal(barrier, inc=1,
                            device_id=(lax.rem(my + k, N_DEV),),
                            device_id_type=pl.DeviceIdType.MESH)

    row_blk = lax.broadcasted_iota(jnp.int32, (SQ, SQ), 0) // 64
    col_blk = lax.broadcasted_iota(jnp.int32, (SQ, SQ), 1) // 64
    mask = (row_blk == col_blk) | (
        lax.rem(col_blk, 4) == lax.rem(row_blk, 4))

    wq_bf = wq_ref[:]
    x2 = x_ref[:].reshape(ROWS, D_MODEL)
    q_all = lax.dot_general(x2, wq_bf, (((1,), (0,)), ((), ())),
                            preferred_element_type=jnp.float32)
    q_all = q_all.astype(jnp.bfloat16)

    for b in range(B):
        part = jnp.zeros((SQ, D_MODEL), jnp.float32)
        for h in range(HQ_LOCAL):
            q = q_all[b * SQ:(b + 1) * SQ, h * DH:(h + 1) * DH]
            kh = k_ref[b, :, h * DH:(h + 1) * DH]
            s = lax.dot_general(q, kh, (((1,), (1,)), ((), ())),
                                preferred_element_type=jnp.float32) * 0.125
            s = jnp.where(mask, s, -1e9)
            m = jnp.max(s, axis=1, keepdims=True)
            e = jnp.exp(s - m)
            w = (e / jnp.sum(e, axis=1, keepdims=True)).astype(jnp.bfloat16)
            vh = v_ref[b, :, h * DH:(h + 1) * DH]
            ctx = lax.dot_general(w, vh, (((1,), (0,)), ((), ())),
                                  preferred_element_type=jnp.float32
                                  ).astype(jnp.bfloat16)
            wo_h = wo_ref[h * DH:(h + 1) * DH, :]
            part = part + lax.dot_general(
                ctx, wo_h, (((1,), (0,)), ((), ())),
                preferred_element_type=jnp.float32)
        part_ref[b * SQ:(b + 1) * SQ, :] = part.astype(jnp.bfloat16)

    pl.semaphore_wait(barrier, N_DEV - 1)

    sends_1 = []
    for k in range(1, N_DEV):
        peer = lax.rem(my + k, N_DEV)
        rdma = pltpu.make_async_remote_copy(
            src_ref=part_ref.at[pl.ds(peer * SH, SH)],
            dst_ref=rs_ref.at[my],
            send_sem=send1.at[k - 1],
            recv_sem=recv1.at[my],
            device_id=(peer,),
            device_id_type=pl.DeviceIdType.MESH,
        )
        rdma.start()
        sends_1.append(rdma)
    for k in range(1, N_DEV):
        src = lax.rem(my + k, N_DEV)
        pltpu.make_async_remote_copy(
            src_ref=part_ref.at[pl.ds(src * SH, SH)],
            dst_ref=rs_ref.at[src],
            send_sem=send1.at[k - 1],
            recv_sem=recv1.at[src],
            device_id=(src,),
            device_id_type=pl.DeviceIdType.MESH,
        ).wait_recv()

    rs_ref[my] = part_ref[pl.ds(my * SH, SH), :]
    red = jnp.zeros((SH, D_MODEL), jnp.float32)
    for o in range(N_DEV):
        red = red + rs_ref[o].astype(jnp.float32)
    red_ref[...] = red.astype(jnp.bfloat16)
    ag_ref[my] = red_ref[...]

    sends_2 = []
    for k in range(1, N_DEV):
        peer = lax.rem(my + k, N_DEV)
        rdma = pltpu.make_async_remote_copy(
            src_ref=red_ref,
            dst_ref=ag_ref.at[my],
            send_sem=send2.at[k - 1],
            recv_sem=recv2.at[my],
            device_id=(peer,),
            device_id_type=pl.DeviceIdType.MESH,
        )
        rdma.start()
        sends_2.append(rdma)
    for k in range(1, N_DEV):
        src = lax.rem(my + k, N_DEV)
        pltpu.make_async_remote_copy(
            src_ref=red_ref,
            dst_ref=ag_ref.at[src],
            send_sem=send2.at[k - 1],
            recv_sem=recv2.at[src],
            device_id=(src,),
            device_id_type=pl.DeviceIdType.MESH,
        ).wait_recv()

    for s in range(N_DEV):
        out_ref[s * SH // SQ, (s * SH) % SQ:(s * SH) % SQ + SH, :] = (
            ag_ref[s].astype(jnp.float32))

    for rdma in sends_1 + sends_2:
        rdma.wait_send()


def kernel(x, Wq, K_ext, V_ext, Wo):
    my = lax.axis_index("i")
    wq_s = lax.dynamic_slice(Wq, (0, my * D_LOC),
                             (D_MODEL, D_LOC)).astype(jnp.bfloat16)
    wo_s = lax.dynamic_slice(Wo, (my * D_LOC, 0),
                             (D_LOC, D_MODEL)).astype(jnp.bfloat16)
    x16 = x.astype(jnp.bfloat16)
    k2 = K_ext.astype(jnp.bfloat16).reshape(B, SQ, D_LOC)
    v2 = V_ext.astype(jnp.bfloat16).reshape(B, SQ, D_LOC)

    return pl.pallas_call(
        _body,
        out_shape=jax.ShapeDtypeStruct((B, SQ, D_MODEL), jnp.float32),
        in_specs=[pl.BlockSpec(memory_space=pltpu.VMEM)] * 5,
        out_specs=pl.BlockSpec(memory_space=pltpu.VMEM),
        scratch_shapes=[
            pltpu.VMEM((ROWS, D_MODEL), jnp.bfloat16),
            pltpu.VMEM((N_DEV, SH, D_MODEL), jnp.bfloat16),
            pltpu.VMEM((SH, D_MODEL), jnp.bfloat16),
            pltpu.VMEM((N_DEV, SH, D_MODEL), jnp.bfloat16),
            pltpu.SemaphoreType.DMA((N_DEV - 1,)),
            pltpu.SemaphoreType.DMA((N_DEV,)),
            pltpu.SemaphoreType.DMA((N_DEV - 1,)),
            pltpu.SemaphoreType.DMA((N_DEV,)),
        ],
        compiler_params=pltpu.CompilerParams(collective_id=0),
    )(x16, k2, v2, wq_s, wo_s)


# device time: 18352 ns/iter; 1.1821x vs baseline; 1.0052x over previous
import jax
import jax.numpy as jnp
from jax import lax
from jax.experimental import pallas as pl
from jax.experimental.pallas import tpu as pltpu

N_DEV = 8
B = 2
SQ = 128
HQ_LOCAL = 4
DH = 64
D_MODEL = 512
D_LOC = HQ_LOCAL * DH
ROWS = B * SQ
SH = ROWS // N_DEV


def _body(x_ref, k_ref, v_ref, wq_ref, wo_ref, out_ref,
          part_ref, rs_ref, red_ref, ag_ref, out_vmem,
          send1, recv1, send2, recv2, out_sem):
    my = lax.axis_index("i")

    barrier = pltpu.get_barrier_semaphore()
    for k in range(1, N_DEV):
        pl.semaphore_signal(barrier, inc=1,
                            device_id=(lax.rem(my + k, N_DEV),),
                            device_id_type=pl.DeviceIdType.MESH)

    row_blk = lax.broadcasted_iota(jnp.int32, (SQ, SQ), 0) // 64
    col_blk = lax.broadcasted_iota(jnp.int32, (SQ, SQ), 1) // 64
    mask = (row_blk == col_blk) | (
        lax.rem(col_blk, 4) == lax.rem(row_blk, 4))

    wq_bf = wq_ref[:]
    x2 = x_ref[:].reshape(ROWS, D_MODEL).astype(jnp.bfloat16)
    q_all = lax.dot_general(x2, wq_bf, (((1,), (0,)), ((), ())),
                            preferred_element_type=jnp.float32)
    q_all = q_all.astype(jnp.bfloat16)

    def attn_batch(b):
        part = jnp.zeros((SQ, D_MODEL), jnp.float32)
        for h in range(HQ_LOCAL):
            q = q_all[b * SQ:(b + 1) * SQ, h * DH:(h + 1) * DH]
            kh = k_ref[b, :, h * DH:(h + 1) * DH]
            s = lax.dot_general(q, kh, (((1,), (1,)), ((), ())),
                                preferred_element_type=jnp.float32) * 0.125
            s = jnp.where(mask, s, -1e9)
            m = jnp.max(s, axis=1, keepdims=True)
            e = jnp.exp(s - m)
            w = (e / jnp.sum(e, axis=1, keepdims=True)).astype(jnp.bfloat16)
            vh = v_ref[b, :, h * DH:(h + 1) * DH]
            ctx = lax.dot_general(w, vh, (((1,), (0,)), ((), ())),
                                  preferred_element_type=jnp.float32
                                  ).astype(jnp.bfloat16)
            wo_h = wo_ref[h * DH:(h + 1) * DH, :]
            part = part + lax.dot_general(
                ctx, wo_h, (((1,), (0,)), ((), ())),
                preferred_element_type=jnp.float32)
        part_ref[b * SQ:(b + 1) * SQ, :] = part.astype(jnp.bfloat16)

    sends_1 = []
    for k in range(1, N_DEV):
        peer = lax.rem(my + k, N_DEV)
        sends_1.append((peer, pltpu.make_async_remote_copy(
            src_ref=part_ref.at[pl.ds(peer * SH, SH)],
            dst_ref=rs_ref.at[my],
            send_sem=send1.at[k - 1],
            recv_sem=recv1.at[my],
            device_id=(peer,),
            device_id_type=pl.DeviceIdType.MESH,
        )))

    attn_batch(0)
    attn_batch(1)
    pl.semaphore_wait(barrier, N_DEV - 1)
    sends_1 = [r for _, r in sends_1]
    for rdma in sends_1:
        rdma.start()
    for k in range(1, N_DEV):
        src = lax.rem(my + k, N_DEV)
        pltpu.make_async_remote_copy(
            src_ref=part_ref.at[pl.ds(src * SH, SH)],
            dst_ref=rs_ref.at[src],
            send_sem=send1.at[k - 1],
            recv_sem=recv1.at[src],
            device_id=(src,),
            device_id_type=pl.DeviceIdType.MESH,
        ).wait_recv()

    rs_ref[my] = part_ref[pl.ds(my * SH, SH), :]
    red = jnp.zeros((SH, D_MODEL), jnp.float32)
    for o in range(N_DEV):
        red = red + rs_ref[o].astype(jnp.float32)
    red_ref[...] = red.astype(jnp.bfloat16)
    ag_ref[my] = red_ref[...]

    sends_2 = []
    for k in range(1, N_DEV):
        peer = lax.rem(my + k, N_DEV)
        rdma = pltpu.make_async_remote_copy(
            src_ref=red_ref,
            dst_ref=ag_ref.at[my],
            send_sem=send2.at[k - 1],
            recv_sem=recv2.at[my],
            device_id=(peer,),
            device_id_type=pl.DeviceIdType.MESH,
        )
        rdma.start()
        sends_2.append(rdma)
    for k in range(1, N_DEV):
        src = lax.rem(my + k, N_DEV)
        pltpu.make_async_remote_copy(
            src_ref=red_ref,
            dst_ref=ag_ref.at[src],
            send_sem=send2.at[k - 1],
            recv_sem=recv2.at[src],
            device_id=(src,),
            device_id_type=pl.DeviceIdType.MESH,
        ).wait_recv()

    for s in range(N_DEV):
        out_vmem[s * SH // SQ, (s * SH) % SQ:(s * SH) % SQ + SH, :] = (
            ag_ref[s].astype(jnp.float32))
    cp_out = pltpu.make_async_copy(out_vmem, out_ref, out_sem.at[0])
    cp_out.start()
    cp_out.wait()

    for rdma in sends_1 + sends_2:
        rdma.wait_send()


def kernel(x, Wq, K_ext, V_ext, Wo):
    my = lax.axis_index("i")
    wq_s = lax.dynamic_slice(Wq, (0, my * D_LOC),
                             (D_MODEL, D_LOC)).astype(jnp.bfloat16)
    wo_s = lax.dynamic_slice(Wo, (my * D_LOC, 0),
                             (D_LOC, D_MODEL)).astype(jnp.bfloat16)
    k2 = K_ext.astype(jnp.bfloat16).reshape(B, SQ, D_LOC)
    v2 = V_ext.astype(jnp.bfloat16).reshape(B, SQ, D_LOC)

    return pl.pallas_call(
        _body,
        out_shape=jax.ShapeDtypeStruct((B, SQ, D_MODEL), jnp.float32),
        in_specs=[pl.BlockSpec(memory_space=pltpu.VMEM)] * 5,
        out_specs=pl.BlockSpec(memory_space=pltpu.MemorySpace.HBM),
        scratch_shapes=[
            pltpu.VMEM((ROWS, D_MODEL), jnp.bfloat16),
            pltpu.VMEM((N_DEV, SH, D_MODEL), jnp.bfloat16),
            pltpu.VMEM((SH, D_MODEL), jnp.bfloat16),
            pltpu.VMEM((N_DEV, SH, D_MODEL), jnp.bfloat16),
            pltpu.VMEM((B, SQ, D_MODEL), jnp.float32),
            pltpu.SemaphoreType.DMA((N_DEV - 1,)),
            pltpu.SemaphoreType.DMA((N_DEV,)),
            pltpu.SemaphoreType.DMA((N_DEV - 1,)),
            pltpu.SemaphoreType.DMA((N_DEV,)),
            pltpu.SemaphoreType.DMA((1,)),
        ],
        compiler_params=pltpu.CompilerParams(collective_id=0),
    )(x, k2, v2, wq_s, wo_s)


# device time: 18254 ns/iter; 1.1885x vs baseline; 1.0054x over previous
import jax
import jax.numpy as jnp
from jax import lax
from jax.experimental import pallas as pl
from jax.experimental.pallas import tpu as pltpu

N_DEV = 8
B = 2
SQ = 128
HQ_LOCAL = 4
DH = 64
D_MODEL = 512
D_LOC = HQ_LOCAL * DH
ROWS = B * SQ
SH = ROWS // N_DEV


def _body(x_ref, k_ref, v_ref, wq_ref, wo_ref, out_ref,
          part_ref, rs_ref, red_ref, ag_ref,
          send1, recv1, send2, recv2):
    my = lax.axis_index("i")

    barrier = pltpu.get_barrier_semaphore()
    for k in range(1, N_DEV):
        pl.semaphore_signal(barrier, inc=1,
                            device_id=(lax.rem(my + k, N_DEV),),
                            device_id_type=pl.DeviceIdType.MESH)

    row_blk = lax.broadcasted_iota(jnp.int32, (SQ, SQ), 0) // 64
    col_blk = lax.broadcasted_iota(jnp.int32, (SQ, SQ), 1) // 64
    mask = (row_blk == col_blk) | (
        lax.rem(col_blk, 4) == lax.rem(row_blk, 4))

    wq_bf = wq_ref[:]
    x2 = x_ref[:].reshape(ROWS, D_MODEL).astype(jnp.bfloat16)
    q_all = lax.dot_general(x2, wq_bf, (((1,), (0,)), ((), ())),
                            preferred_element_type=jnp.float32)
    q_all = q_all.astype(jnp.bfloat16)

    def attn_batch(b):
        part = jnp.zeros((SQ, D_MODEL), jnp.float32)
        for h in range(HQ_LOCAL):
            q = q_all[b * SQ:(b + 1) * SQ, h * DH:(h + 1) * DH]
            kh = k_ref[b, :, h * DH:(h + 1) * DH]
            s = lax.dot_general(q, kh, (((1,), (1,)), ((), ())),
                                preferred_element_type=jnp.float32) * 0.125
            s = jnp.where(mask, s, -1e9)
            m = jnp.max(s, axis=1, keepdims=True)
            e = jnp.exp(s - m)
            w = (e / jnp.sum(e, axis=1, keepdims=True)).astype(jnp.bfloat16)
            vh = v_ref[b, :, h * DH:(h + 1) * DH]
            ctx = lax.dot_general(w, vh, (((1,), (0,)), ((), ())),
                                  preferred_element_type=jnp.float32
                                  ).astype(jnp.bfloat16)
            wo_h = wo_ref[h * DH:(h + 1) * DH, :]
            part = part + lax.dot_general(
                ctx, wo_h, (((1,), (0,)), ((), ())),
                preferred_element_type=jnp.float32)
        part_ref[b * SQ:(b + 1) * SQ, :] = part.astype(jnp.bfloat16)

    sends_1 = []
    for k in range(1, N_DEV):
        peer = lax.rem(my + k, N_DEV)
        sends_1.append((peer, pltpu.make_async_remote_copy(
            src_ref=part_ref.at[pl.ds(peer * SH, SH)],
            dst_ref=rs_ref.at[my],
            send_sem=send1.at[k - 1],
            recv_sem=recv1.at[my],
            device_id=(peer,),
            device_id_type=pl.DeviceIdType.MESH,
        )))

    attn_batch(0)
    attn_batch(1)
    pl.semaphore_wait(barrier, N_DEV - 1)
    sends_1 = [r for _, r in sends_1]
    for rdma in sends_1:
        rdma.start()
    for k in range(1, N_DEV):
        src = lax.rem(my + k, N_DEV)
        pltpu.make_async_remote_copy(
            src_ref=part_ref.at[pl.ds(src * SH, SH)],
            dst_ref=rs_ref.at[src],
            send_sem=send1.at[k - 1],
            recv_sem=recv1.at[src],
            device_id=(src,),
            device_id_type=pl.DeviceIdType.MESH,
        ).wait_recv()

    rs_ref[my] = part_ref[pl.ds(my * SH, SH), :]
    red = jnp.zeros((SH, D_MODEL), jnp.float32)
    for o in range(N_DEV):
        red = red + rs_ref[o].astype(jnp.float32)
    red_ref[...] = red.astype(jnp.bfloat16)
    ag_ref[my] = red_ref[...]

    sends_2 = []
    for k in range(1, N_DEV):
        peer = lax.rem(my + k, N_DEV)
        rdma = pltpu.make_async_remote_copy(
            src_ref=red_ref,
            dst_ref=ag_ref.at[my],
            send_sem=send2.at[k - 1],
            recv_sem=recv2.at[my],
            device_id=(peer,),
            device_id_type=pl.DeviceIdType.MESH,
        )
        rdma.start()
        sends_2.append(rdma)
    for k in range(1, N_DEV):
        src = lax.rem(my + k, N_DEV)
        pltpu.make_async_remote_copy(
            src_ref=red_ref,
            dst_ref=ag_ref.at[src],
            send_sem=send2.at[k - 1],
            recv_sem=recv2.at[src],
            device_id=(src,),
            device_id_type=pl.DeviceIdType.MESH,
        ).wait_recv()

    for s in range(N_DEV):
        out_ref[s * SH // SQ, (s * SH) % SQ:(s * SH) % SQ + SH, :] = (
            ag_ref[s].astype(jnp.float32))

    for rdma in sends_1 + sends_2:
        rdma.wait_send()


def kernel(x, Wq, K_ext, V_ext, Wo):
    my = lax.axis_index("i")
    wq_s = lax.dynamic_slice(Wq, (0, my * D_LOC),
                             (D_MODEL, D_LOC)).astype(jnp.bfloat16)
    wo_s = lax.dynamic_slice(Wo, (my * D_LOC, 0),
                             (D_LOC, D_MODEL)).astype(jnp.bfloat16)
    k2 = K_ext.astype(jnp.bfloat16).reshape(B, SQ, D_LOC)
    v2 = V_ext.astype(jnp.bfloat16).reshape(B, SQ, D_LOC)

    return pl.pallas_call(
        _body,
        out_shape=jax.ShapeDtypeStruct((B, SQ, D_MODEL), jnp.float32),
        in_specs=[pl.BlockSpec(memory_space=pltpu.VMEM)] * 5,
        out_specs=pl.BlockSpec(memory_space=pltpu.VMEM),
        scratch_shapes=[
            pltpu.VMEM((ROWS, D_MODEL), jnp.bfloat16),
            pltpu.VMEM((N_DEV, SH, D_MODEL), jnp.bfloat16),
            pltpu.VMEM((SH, D_MODEL), jnp.bfloat16),
            pltpu.VMEM((N_DEV, SH, D_MODEL), jnp.bfloat16),
            pltpu.SemaphoreType.DMA((N_DEV - 1,)),
            pltpu.SemaphoreType.DMA((N_DEV,)),
            pltpu.SemaphoreType.DMA((N_DEV - 1,)),
            pltpu.SemaphoreType.DMA((N_DEV,)),
        ],
        compiler_params=pltpu.CompilerParams(collective_id=0),
    )(x, k2, v2, wq_s, wo_s)


# device time: 18006 ns/iter; 1.2048x vs baseline; 1.0138x over previous
import jax
import jax.numpy as jnp
from jax import lax
from jax.experimental import pallas as pl
from jax.experimental.pallas import tpu as pltpu

N_DEV = 8
B = 2
SQ = 128
HQ_LOCAL = 4
DH = 64
D_MODEL = 512
D_LOC = HQ_LOCAL * DH
ROWS = B * SQ
SH = ROWS // N_DEV
SH2 = SH // 2


def _body(x_ref, k_ref, v_ref, wq_ref, wo_ref, out_ref,
          part_ref, rs_ref, red_ref, ag_ref,
          send1, recv1, send2, recv2):
    my = lax.axis_index("i")

    barrier = pltpu.get_barrier_semaphore()
    for k in range(1, N_DEV):
        pl.semaphore_signal(barrier, inc=1,
                            device_id=(lax.rem(my + k, N_DEV),),
                            device_id_type=pl.DeviceIdType.MESH)

    row_blk = lax.broadcasted_iota(jnp.int32, (SQ, SQ), 0) // 64
    col_blk = lax.broadcasted_iota(jnp.int32, (SQ, SQ), 1) // 64
    mask = (row_blk == col_blk) | (
        lax.rem(col_blk, 4) == lax.rem(row_blk, 4))

    wq_bf = wq_ref[:]
    x2 = x_ref[:].reshape(ROWS, D_MODEL).astype(jnp.bfloat16)
    q_all = lax.dot_general(x2, wq_bf, (((1,), (0,)), ((), ())),
                            preferred_element_type=jnp.float32)
    q_all = q_all.astype(jnp.bfloat16)

    def attn_batch(b):
        part = jnp.zeros((SQ, D_MODEL), jnp.float32)
        for h in range(HQ_LOCAL):
            q = q_all[b * SQ:(b + 1) * SQ, h * DH:(h + 1) * DH]
            kh = k_ref[b, :, h * DH:(h + 1) * DH]
            s = lax.dot_general(q, kh, (((1,), (1,)), ((), ())),
                                preferred_element_type=jnp.float32) * 0.125
            s = jnp.where(mask, s, -1e9)
            m = jnp.max(s, axis=1, keepdims=True)
            e = jnp.exp(s - m)
            w = (e / jnp.sum(e, axis=1, keepdims=True)).astype(jnp.bfloat16)
            vh = v_ref[b, :, h * DH:(h + 1) * DH]
            ctx = lax.dot_general(w, vh, (((1,), (0,)), ((), ())),
                                  preferred_element_type=jnp.float32
                                  ).astype(jnp.bfloat16)
            wo_h = wo_ref[h * DH:(h + 1) * DH, :]
            part = part + lax.dot_general(
                ctx, wo_h, (((1,), (0,)), ((), ())),
                preferred_element_type=jnp.float32)
        part_ref[b * SQ:(b + 1) * SQ, :] = part.astype(jnp.bfloat16)

    attn_batch(0)
    attn_batch(1)
    pl.semaphore_wait(barrier, N_DEV - 1)

    sends = []
    for t in range(2):
        for k in range(1, N_DEV):
            peer = lax.rem(my + k, N_DEV)
            rdma = pltpu.make_async_remote_copy(
                src_ref=part_ref.at[pl.ds(peer * SH + t * SH2, SH2)],
                dst_ref=rs_ref.at[t, my],
                send_sem=send1.at[t, k - 1],
                recv_sem=recv1.at[t, my],
                device_id=(peer,),
                device_id_type=pl.DeviceIdType.MESH,
            )
            rdma.start()
            sends.append(rdma)

    for t in range(2):
        for k in range(1, N_DEV):
            src_d = lax.rem(my + k, N_DEV)
            pltpu.make_async_remote_copy(
                src_ref=part_ref.at[pl.ds(src_d * SH + t * SH2, SH2)],
                dst_ref=rs_ref.at[t, src_d],
                send_sem=send1.at[t, k - 1],
                recv_sem=recv1.at[t, src_d],
                device_id=(src_d,),
                device_id_type=pl.DeviceIdType.MESH,
            ).wait_recv()
        rs_ref[t, my] = part_ref[pl.ds(my * SH + t * SH2, SH2), :]
        red = jnp.zeros((SH2, D_MODEL), jnp.float32)
        for o in range(N_DEV):
            red = red + rs_ref[t, o].astype(jnp.float32)
        red_ref[t] = red.astype(jnp.bfloat16)
        ag_ref[t, my] = red_ref[t]
        for k in range(1, N_DEV):
            peer = lax.rem(my + k, N_DEV)
            rdma = pltpu.make_async_remote_copy(
                src_ref=red_ref.at[t],
                dst_ref=ag_ref.at[t, my],
                send_sem=send2.at[t, k - 1],
                recv_sem=recv2.at[t, my],
                device_id=(peer,),
                device_id_type=pl.DeviceIdType.MESH,
            )
            rdma.start()
            sends.append(rdma)

    for t in range(2):
        for k in range(1, N_DEV):
            src_d = lax.rem(my + k, N_DEV)
            pltpu.make_async_remote_copy(
                src_ref=red_ref.at[t],
                dst_ref=ag_ref.at[t, src_d],
                send_sem=send2.at[t, k - 1],
                recv_sem=recv2.at[t, src_d],
                device_id=(src_d,),
                device_id_type=pl.DeviceIdType.MESH,
            ).wait_recv()

    for s in range(N_DEV):
        for t in range(2):
            r0 = (s * SH) % SQ + t * SH2
            out_ref[s * SH // SQ, r0:r0 + SH2, :] = (
                ag_ref[t, s].astype(jnp.float32))

    for rdma in sends:
        rdma.wait_send()


def kernel(x, Wq, K_ext, V_ext, Wo):
    my = lax.axis_index("i")
    wq_s = lax.dynamic_slice(Wq, (0, my * D_LOC),
                             (D_MODEL, D_LOC)).astype(jnp.bfloat16)
    wo_s = lax.dynamic_slice(Wo, (my * D_LOC, 0),
                             (D_LOC, D_MODEL)).astype(jnp.bfloat16)
    k2 = K_ext.astype(jnp.bfloat16).reshape(B, SQ, D_LOC)
    v2 = V_ext.astype(jnp.bfloat16).reshape(B, SQ, D_LOC)

    return pl.pallas_call(
        _body,
        out_shape=jax.ShapeDtypeStruct((B, SQ, D_MODEL), jnp.float32),
        in_specs=[pl.BlockSpec(memory_space=pltpu.VMEM)] * 5,
        out_specs=pl.BlockSpec(memory_space=pltpu.VMEM),
        scratch_shapes=[
            pltpu.VMEM((ROWS, D_MODEL), jnp.bfloat16),
            pltpu.VMEM((2, N_DEV, SH2, D_MODEL), jnp.bfloat16),
            pltpu.VMEM((2, SH2, D_MODEL), jnp.bfloat16),
            pltpu.VMEM((2, N_DEV, SH2, D_MODEL), jnp.bfloat16),
            pltpu.SemaphoreType.DMA((2, N_DEV - 1)),
            pltpu.SemaphoreType.DMA((2, N_DEV)),
            pltpu.SemaphoreType.DMA((2, N_DEV - 1)),
            pltpu.SemaphoreType.DMA((2, N_DEV)),
        ],
        compiler_params=pltpu.CompilerParams(collective_id=0),
    )(x, k2, v2, wq_s, wo_s)
